# baseline (speedup 1.0000x reference)
import functools

import jax
import jax.numpy as jnp
from jax import lax
from jax.experimental import pallas as pl
from jax.experimental.pallas import tpu as pltpu
from jax.experimental.pallas import tpu_sc as plsc

VOCAB = 1440
EMBED = 48
NC = 2
NS = 16
NW = NC * NS
LANE = 16
IBW = 128
NB = 4
BATCH = NB * IBW
K8 = EMBED // 8
TSTRIDE = IBW + 1


@functools.lru_cache(maxsize=None)
def _build(n_i: int, n_j: int):
    n_batches = (n_i // IBW // NB) * n_j
    per_w = n_batches // NW
    assert per_w % 2 == 0
    bat_per_j = n_i // BATCH
    mesh = plsc.VectorSubcoreMesh(
        core_axis_name="c", subcore_axis_name="s",
        num_cores=NC, num_subcores=NS)

    @functools.partial(
        pl.kernel,
        out_type=jax.ShapeDtypeStruct((n_j, K8, n_i // IBW, 8, IBW),
                                      jnp.float32),
        mesh=mesh,
        scratch_types=[
            pltpu.VMEM((VOCAB * EMBED,), jnp.float32),
            pltpu.VMEM((2, BATCH), jnp.int32),
            pltpu.VMEM((2, NB, EMBED, TSTRIDE), jnp.float32),
            pltpu.SemaphoreType.DMA,
            pltpu.SemaphoreType.DMA,
            pltpu.SemaphoreType.DMA,
            pltpu.SemaphoreType.DMA,
        ],
        compiler_params=pltpu.CompilerParams(
            use_tc_tiling_on_sc=False, needs_layout_passes=False),
    )
    def gather_kernel(table_hbm, xt_hbm, out_hbm, table_v, idx_v, trans_v,
                      si0, si1, so0, so1):
        sem_i = (si0, si1)
        sem_o = (so0, so1)
        wid = lax.axis_index("s") * NC + lax.axis_index("c")
        b0 = wid * per_w

        pltpu.sync_copy(table_hbm, table_v)

        iota = lax.iota(jnp.int32, LANE)
        kadd = [m * LANE + iota for m in range(EMBED // LANE)]
        kvec = kadd

        def lane_bcast(vec, l):
            return lax.gather(
                vec, jnp.full((LANE, 1), l, jnp.int32),
                lax.GatherDimensionNumbers(
                    offset_dims=(), collapsed_slice_dims=(0,),
                    start_index_map=(0,)),
                slice_sizes=(1,),
                mode=lax.GatherScatterMode.PROMISE_IN_BOUNDS)

        def start_idx(b, slot):
            j = b // bat_per_j
            col = (b % bat_per_j) * BATCH
            pltpu.async_copy(
                xt_hbm.at[j, pl.ds(pl.multiple_of(col, 8), BATCH)],
                idx_v.at[slot], sem_i[slot])

        def wait_idx(slot):
            pltpu.make_async_copy(
                xt_hbm.at[0, pl.ds(0, BATCH)], idx_v.at[slot],
                sem_i[slot]).wait()

        def wait_out(slot):
            for k8 in range(K8):
                pltpu.make_async_copy(
                    trans_v.at[slot, :, pl.ds(k8 * 8, 8), pl.ds(0, IBW)],
                    out_hbm.at[0, k8, pl.ds(0, NB)],
                    sem_o[slot]).wait()

        start_idx(b0, 0)
        start_idx(b0 + 1, 1)

        @pl.loop(0, per_w, step=2)
        def _batches(g):
            for s in range(2):
                b = b0 + g + s
                j = b // bat_per_j
                ib0 = (b % bat_per_j) * NB

                @pl.when(g >= 2)
                def _():
                    wait_out(s)

                wait_idx(s)
                for ib in range(NB):
                    ibv = jnp.full((LANE,), ib, jnp.int32)

                    @plsc.parallel_loop(0, IBW // LANE)
                    def _groups(gg):
                        off = ib * IBW + gg * LANE
                        idxv = idx_v[s, pl.ds(off, LANE)]
                        base = idxv * EMBED
                        goff = gg * LANE
                        for l in range(LANE):
                            bl = lane_bcast(base, l)
                            ilv = jnp.full((LANE,), goff + l, jnp.int32)
                            for m in range(EMBED // LANE):
                                val = plsc.load_gather(
                                    table_v, [bl + kadd[m]])
                                plsc.store_scatter(
                                    trans_v.at[s], [ibv, kvec[m], ilv],
                                    val)

                for k8 in range(K8):
                    pltpu.async_copy(
                        trans_v.at[s, :, pl.ds(k8 * 8, 8), pl.ds(0, IBW)],
                        out_hbm.at[j, k8,
                                   pl.ds(pl.multiple_of(ib0, 4), NB)],
                        sem_o[s])

                @pl.when(g < per_w - 2)
                def _():
                    start_idx(b + 2, s)

        for s in range(2):
            wait_out(s)

    return gather_kernel


def kernel(x, table):
    n_i, n_j = x.shape
    xt = x.T.astype(jnp.int32)
    tab = table.astype(jnp.float32).reshape(-1)
    out5 = _build(n_i, n_j)(tab, xt)
    out = out5.transpose(2, 4, 0, 1, 3).reshape(n_i, n_j, EMBED)
    return out


# speedup vs baseline: 1.1115x; 1.1115x over previous
import functools

import jax
import jax.numpy as jnp
from jax import lax
from jax.experimental import pallas as pl
from jax.experimental.pallas import tpu as pltpu
from jax.experimental.pallas import tpu_sc as plsc

VOCAB = 1440
EMBED = 48
NC = 2
NS = 16
NW = NC * NS
LANE = 16
IBW = 128
NB = 4
BATCH = NB * IBW
K8 = EMBED // 8
TSTRIDE = IBW + 1


@functools.lru_cache(maxsize=None)
def _build(n_i: int, n_j: int):
    n_batches = (n_i // IBW // NB) * n_j
    per_w = n_batches // NW
    assert per_w % 2 == 0
    bat_per_j = n_i // BATCH
    mesh = plsc.VectorSubcoreMesh(
        core_axis_name="c", subcore_axis_name="s",
        num_cores=NC, num_subcores=NS)

    @functools.partial(
        pl.kernel,
        out_type=jax.ShapeDtypeStruct((n_j, K8, n_i // IBW, 8, IBW),
                                      jnp.float32),
        mesh=mesh,
        scratch_types=[
            pltpu.VMEM((VOCAB * EMBED,), jnp.float32),
            pltpu.VMEM((2, BATCH), jnp.int32),
            pltpu.VMEM((2, NB, EMBED, TSTRIDE), jnp.float32),
            pltpu.SemaphoreType.DMA,
            pltpu.SemaphoreType.DMA,
            pltpu.SemaphoreType.DMA,
            pltpu.SemaphoreType.DMA,
        ],
        compiler_params=pltpu.CompilerParams(
            use_tc_tiling_on_sc=False, needs_layout_passes=False),
    )
    def gather_kernel(table_hbm, xt_hbm, out_hbm, table_v, idx_v, trans_v,
                      si0, si1, so0, so1):
        sem_i = (si0, si1)
        sem_o = (so0, so1)
        wid = lax.axis_index("s") * NC + lax.axis_index("c")
        b0 = wid * per_w

        pltpu.sync_copy(table_hbm, table_v)

        iota = lax.iota(jnp.int32, LANE)
        kvec = [m * LANE + iota for m in range(EMBED // LANE)]

        def start_idx(b, slot):
            j = b // bat_per_j
            col = (b % bat_per_j) * BATCH
            pltpu.async_copy(
                xt_hbm.at[j, pl.ds(pl.multiple_of(col, 8), BATCH)],
                idx_v.at[slot], sem_i[slot])

        def wait_idx(slot):
            pltpu.make_async_copy(
                xt_hbm.at[0, pl.ds(0, BATCH)], idx_v.at[slot],
                sem_i[slot]).wait()

        def wait_out(slot):
            for k8 in range(K8):
                pltpu.make_async_copy(
                    trans_v.at[slot, :, pl.ds(k8 * 8, 8), pl.ds(0, IBW)],
                    out_hbm.at[0, k8, pl.ds(0, NB)],
                    sem_o[slot]).wait()

        start_idx(b0, 0)
        start_idx(b0 + 1, 1)

        @pl.loop(0, per_w, step=2)
        def _batches(g):
            for s in range(2):
                b = b0 + g + s
                j = b // bat_per_j
                ib0 = (b % bat_per_j) * NB

                @pl.when(g >= 2)
                def _():
                    wait_out(s)

                wait_idx(s)
                for ib in range(NB):
                    ibv = jnp.full((LANE,), ib, jnp.int32)

                    @plsc.parallel_loop(0, IBW // LANE)
                    def _groups(gg):
                        off = ib * IBW + gg * LANE
                        idxv = idx_v[s, pl.ds(off, LANE)]
                        basev = idxv * EMBED
                        goff = gg * LANE
                        for l in range(LANE):
                            base = basev[l]
                            ilv = jnp.full((LANE,), goff + l, jnp.int32)
                            for m in range(EMBED // LANE):
                                val = table_v[pl.ds(base + m * LANE,
                                                    LANE)]
                                plsc.store_scatter(
                                    trans_v.at[s], [ibv, kvec[m], ilv],
                                    val)

                for k8 in range(K8):
                    pltpu.async_copy(
                        trans_v.at[s, :, pl.ds(k8 * 8, 8), pl.ds(0, IBW)],
                        out_hbm.at[j, k8,
                                   pl.ds(pl.multiple_of(ib0, 4), NB)],
                        sem_o[s])

                @pl.when(g < per_w - 2)
                def _():
                    start_idx(b + 2, s)

        for s in range(2):
            wait_out(s)

    return gather_kernel


def kernel(x, table):
    n_i, n_j = x.shape
    xt = x.T.astype(jnp.int32)
    tab = table.astype(jnp.float32).reshape(-1)
    out5 = _build(n_i, n_j)(tab, xt)
    out = out5.transpose(2, 4, 0, 1, 3).reshape(n_i, n_j, EMBED)
    return out


# speedup vs baseline: 1.1644x; 1.0476x over previous
import functools

import jax
import jax.numpy as jnp
from jax import lax
from jax.experimental import pallas as pl
from jax.experimental.pallas import tpu as pltpu
from jax.experimental.pallas import tpu_sc as plsc

VOCAB = 1440
EMBED = 48
NC = 2
NS = 16
NW = NC * NS
LANE = 16
IBW = 128
NB = 4
BATCH = NB * IBW
K8 = EMBED // 8
TPAD = EMBED + 1


@functools.lru_cache(maxsize=None)
def _build(n_i: int, n_j: int):
    n_batches = (n_i // IBW // NB) * n_j
    per_w = n_batches // NW
    assert per_w % 2 == 0
    bat_per_j = n_i // BATCH
    mesh = plsc.VectorSubcoreMesh(
        core_axis_name="c", subcore_axis_name="s",
        num_cores=NC, num_subcores=NS)

    @functools.partial(
        pl.kernel,
        out_type=jax.ShapeDtypeStruct((n_j, K8, n_i // IBW, 8 * IBW),
                                      jnp.float32),
        mesh=mesh,
        scratch_types=[
            pltpu.VMEM((VOCAB * TPAD,), jnp.float32),
            pltpu.VMEM((2, BATCH), jnp.int32),
            pltpu.VMEM((2, NB, K8, 8 * IBW), jnp.float32),
            pltpu.SemaphoreType.DMA,
            pltpu.SemaphoreType.DMA,
            pltpu.SemaphoreType.DMA,
            pltpu.SemaphoreType.DMA,
        ],
        compiler_params=pltpu.CompilerParams(
            use_tc_tiling_on_sc=False, needs_layout_passes=False),
    )
    def gather_kernel(table_hbm, xt_hbm, out_hbm, table_v, idx_v, trans_v,
                      si0, si1, so0, so1):
        sem_i = (si0, si1)
        sem_o = (so0, so1)
        wid = lax.axis_index("s") * NC + lax.axis_index("c")
        b0 = wid * per_w

        pltpu.sync_copy(table_hbm, table_v)

        def start_idx(b, slot):
            j = b // bat_per_j
            col = (b % bat_per_j) * BATCH
            pltpu.async_copy(
                xt_hbm.at[j, pl.ds(pl.multiple_of(col, 8), BATCH)],
                idx_v.at[slot], sem_i[slot])

        def wait_idx(slot):
            pltpu.make_async_copy(
                xt_hbm.at[0, pl.ds(0, BATCH)], idx_v.at[slot],
                sem_i[slot]).wait()

        def wait_out(slot):
            for k8 in range(K8):
                pltpu.make_async_copy(
                    trans_v.at[slot, :, k8],
                    out_hbm.at[0, k8, pl.ds(0, NB)],
                    sem_o[slot]).wait()

        start_idx(b0, 0)
        start_idx(b0 + 1, 1)

        @pl.loop(0, per_w, step=2)
        def _batches(g):
            for s in range(2):
                b = b0 + g + s
                j = b // bat_per_j
                ib0 = (b % bat_per_j) * NB

                @pl.when(g >= 2)
                def _():
                    wait_out(s)

                wait_idx(s)
                for ib in range(NB):
                    @plsc.parallel_loop(0, IBW // LANE, unroll=2)
                    def _groups(gg):
                        off = ib * IBW + gg * LANE
                        idxv = idx_v[s, pl.ds(off, LANE)]
                        addr = idxv * TPAD
                        for k in range(EMBED):
                            val = plsc.load_gather(table_v, [addr + k])
                            trans_v[s, ib, k // 8,
                                    pl.ds((k % 8) * IBW + gg * LANE,
                                          LANE)] = val

                for k8 in range(K8):
                    pltpu.async_copy(
                        trans_v.at[s, :, k8],
                        out_hbm.at[j, k8,
                                   pl.ds(pl.multiple_of(ib0, 4), NB)],
                        sem_o[s])

                @pl.when(g < per_w - 2)
                def _():
                    start_idx(b + 2, s)

        for s in range(2):
            wait_out(s)

    return gather_kernel


def kernel(x, table):
    n_i, n_j = x.shape
    xt = x.T.astype(jnp.int32)
    tab = jnp.pad(table.astype(jnp.float32),
                  ((0, 0), (0, TPAD - EMBED))).reshape(-1)
    out5 = _build(n_i, n_j)(tab, xt)
    out = (out5.reshape(n_j, K8, n_i // IBW, 8, IBW)
           .transpose(2, 4, 0, 1, 3)
           .reshape(n_i, n_j, EMBED))
    return out


# speedup vs baseline: 1.6397x; 1.4082x over previous
import functools

import jax
import jax.numpy as jnp
from jax import lax
from jax.experimental import pallas as pl
from jax.experimental.pallas import tpu as pltpu
from jax.experimental.pallas import tpu_sc as plsc

VOCAB = 1440
EMBED = 48
NC = 2
NS = 16
NW = NC * NS
LANE = 16
IBW = 128
NB = 4
BATCH = NB * IBW
K8 = EMBED // 8
TPAD = EMBED + 1


@functools.lru_cache(maxsize=None)
def _build(n_i: int, n_j: int):
    n_batches = (n_i // IBW // NB) * n_j
    per_w = n_batches // NW
    assert per_w % 2 == 0
    bat_per_j = n_i // BATCH
    mesh = plsc.VectorSubcoreMesh(
        core_axis_name="c", subcore_axis_name="s",
        num_cores=NC, num_subcores=NS)

    @functools.partial(
        pl.kernel,
        out_type=jax.ShapeDtypeStruct((n_j, K8, n_i // IBW, 8 * IBW),
                                      jnp.float32),
        mesh=mesh,
        scratch_types=[
            pltpu.VMEM((VOCAB * TPAD,), jnp.float32),
            pltpu.VMEM((2, BATCH), jnp.int32),
            pltpu.VMEM((2, NB, K8, 8 * IBW), jnp.float32),
            pltpu.SemaphoreType.DMA,
            pltpu.SemaphoreType.DMA,
            pltpu.SemaphoreType.DMA,
            pltpu.SemaphoreType.DMA,
        ],
        compiler_params=pltpu.CompilerParams(
            use_tc_tiling_on_sc=False, needs_layout_passes=False),
    )
    def gather_kernel(table_hbm, xt_hbm, out_hbm, table_v, idx_v, trans_v,
                      si0, si1, so0, so1):
        sem_i = (si0, si1)
        sem_o = (so0, so1)
        wid = lax.axis_index("s") * NC + lax.axis_index("c")
        b0 = wid * per_w

        pltpu.sync_copy(table_hbm, table_v)

        def start_idx(b, slot):
            j = b // bat_per_j
            col = (b % bat_per_j) * BATCH
            pltpu.async_copy(
                xt_hbm.at[j, pl.ds(pl.multiple_of(col, 8), BATCH)],
                idx_v.at[slot], sem_i[slot])

        def wait_idx(slot):
            pltpu.make_async_copy(
                xt_hbm.at[0, pl.ds(0, BATCH)], idx_v.at[slot],
                sem_i[slot]).wait()

        def wait_out(slot):
            for k8 in range(K8):
                pltpu.make_async_copy(
                    trans_v.at[slot, :, k8],
                    out_hbm.at[0, k8, pl.ds(0, NB)],
                    sem_o[slot]).wait()

        start_idx(b0, 0)
        start_idx(b0 + 1, 1)

        @pl.loop(0, per_w, step=2)
        def _batches(g):
            for s in range(2):
                b = b0 + g + s
                j = b // bat_per_j
                ib0 = (b % bat_per_j) * NB

                @pl.when(g >= 2)
                def _():
                    wait_out(s)

                wait_idx(s)
                for ib in range(NB):
                    @plsc.parallel_loop(0, IBW // LANE)
                    def _groups(gg):
                        off = ib * IBW + gg * LANE
                        idxv = idx_v[s, pl.ds(off, LANE)]
                        addr = idxv * TPAD
                        for k in range(EMBED):
                            val = plsc.load_gather(table_v, [addr + k])
                            trans_v[s, ib, k // 8,
                                    pl.ds((k % 8) * IBW + gg * LANE,
                                          LANE)] = val

                for k8 in range(K8):
                    pltpu.async_copy(
                        trans_v.at[s, :, k8],
                        out_hbm.at[j, k8,
                                   pl.ds(pl.multiple_of(ib0, 4), NB)],
                        sem_o[s])

                @pl.when(g < per_w - 2)
                def _():
                    start_idx(b + 2, s)

        for s in range(2):
            wait_out(s)

    return gather_kernel


def kernel(x, table):
    n_i, n_j = x.shape
    xt = x.T.astype(jnp.int32)
    tab = jnp.pad(table.astype(jnp.float32),
                  ((0, 0), (0, TPAD - EMBED))).reshape(-1)
    out5 = _build(n_i, n_j)(tab, xt)
    out = (out5.reshape(n_j, K8, n_i // IBW, 8, IBW)
           .transpose(2, 4, 0, 1, 3)
           .reshape(n_i, n_j, EMBED))
    return out


# speedup vs baseline: 1.8865x; 1.1505x over previous
import functools

import jax
import jax.numpy as jnp
from jax import lax
from jax.experimental import pallas as pl
from jax.experimental.pallas import tpu as pltpu
from jax.experimental.pallas import tpu_sc as plsc

VOCAB = 1440
EMBED = 48
NC = 2
NS = 16
NW = NC * NS
LANE = 16
IBW = 128
NB = 4
BATCH = NB * IBW
K8 = EMBED // 8
TPAD = EMBED + 1


@functools.lru_cache(maxsize=None)
def _build(n_i: int, n_j: int):
    n_batches = (n_i // IBW // NB) * n_j
    per_w = n_batches // NW
    assert per_w % 2 == 0
    bat_per_j = n_i // BATCH
    mesh = plsc.VectorSubcoreMesh(
        core_axis_name="c", subcore_axis_name="s",
        num_cores=NC, num_subcores=NS)

    @functools.partial(
        pl.kernel,
        out_type=jax.ShapeDtypeStruct((n_j, K8, n_i // IBW, 8 * IBW),
                                      jnp.float32),
        mesh=mesh,
        scratch_types=[
            pltpu.VMEM((VOCAB * TPAD,), jnp.float32),
            pltpu.VMEM((2, BATCH), jnp.int32),
            pltpu.VMEM((2, NB, K8, 8 * IBW), jnp.float32),
            pltpu.SemaphoreType.DMA,
            pltpu.SemaphoreType.DMA,
            pltpu.SemaphoreType.DMA,
            pltpu.SemaphoreType.DMA,
        ],
        compiler_params=pltpu.CompilerParams(
            use_tc_tiling_on_sc=False, needs_layout_passes=False),
    )
    def gather_kernel(table_hbm, xt_hbm, out_hbm, table_v, idx_v, trans_v,
                      si0, si1, so0, so1):
        sem_i = (si0, si1)
        sem_o = (so0, so1)
        wid = lax.axis_index("s") * NC + lax.axis_index("c")
        b0 = wid * per_w

        pltpu.sync_copy(table_hbm, table_v)

        def start_idx(b, slot):
            j = b // bat_per_j
            col = (b % bat_per_j) * BATCH
            pltpu.async_copy(
                xt_hbm.at[j, pl.ds(pl.multiple_of(col, 8), BATCH)],
                idx_v.at[slot], sem_i[slot])

        def wait_idx(slot):
            pltpu.make_async_copy(
                xt_hbm.at[0, pl.ds(0, BATCH)], idx_v.at[slot],
                sem_i[slot]).wait()

        def wait_out(slot):
            for k8 in range(K8):
                pltpu.make_async_copy(
                    trans_v.at[slot, :, k8],
                    out_hbm.at[0, k8, pl.ds(0, NB)],
                    sem_o[slot]).wait()

        start_idx(b0, 0)
        start_idx(b0 + 1, 1)

        @pl.loop(0, per_w, step=2)
        def _batches(g):
            for s in range(2):
                b = b0 + g + s
                j = b // bat_per_j
                ib0 = (b % bat_per_j) * NB

                @pl.when(g >= 2)
                def _():
                    wait_out(s)

                wait_idx(s)
                @plsc.parallel_loop(0, BATCH // LANE)
                def _groups(gg):
                    idxv = idx_v[s, pl.ds(gg * LANE, LANE)]
                    addr = idxv * TPAD
                    ib = gg // (IBW // LANE)
                    goff = (gg % (IBW // LANE)) * LANE
                    for k in range(EMBED):
                        val = plsc.load_gather(table_v, [addr + k])
                        trans_v[s, ib, k // 8,
                                pl.ds((k % 8) * IBW + goff, LANE)] = val

                for k8 in range(K8):
                    pltpu.async_copy(
                        trans_v.at[s, :, k8],
                        out_hbm.at[j, k8,
                                   pl.ds(pl.multiple_of(ib0, 4), NB)],
                        sem_o[s])

                @pl.when(g < per_w - 2)
                def _():
                    start_idx(b + 2, s)

        for s in range(2):
            wait_out(s)

    return gather_kernel


def kernel(x, table):
    n_i, n_j = x.shape
    xt = x.T.astype(jnp.int32)
    tab = jnp.pad(table.astype(jnp.float32),
                  ((0, 0), (0, TPAD - EMBED))).reshape(-1)
    out5 = _build(n_i, n_j)(tab, xt)
    out = (out5.reshape(n_j, K8, n_i // IBW, 8, IBW)
           .transpose(2, 4, 0, 1, 3)
           .reshape(n_i, n_j, EMBED))
    return out
